# baseline (speedup 1.0000x reference)
import functools

import jax
import jax.numpy as jnp
from jax import lax
from jax.experimental import pallas as pl
from jax.experimental.pallas import tpu as pltpu
from jax.experimental.pallas import tpu_sc as plsc

MAX_REL = 32
NUM_EMB = 16
NUM_CLASSES = 2 * MAX_REL + 1 + 1

NC = 2
NS = 16
NW = NC * NS
L = 16

C = 2048
G = 128
NG = C // G


def _bias_body(w_ref, b_ref, t_ref):
    t_ref[...] = w_ref[...] + b_ref[...]


def _fold_bias(W, b):
    return pl.pallas_call(
        _bias_body,
        out_shape=jax.ShapeDtypeStruct((NUM_CLASSES, NUM_EMB), jnp.float32),
    )(W, b.reshape(1, NUM_EMB))


def _make_sc_lookup(n):
    assert n % (NW * C) == 0
    bpw = n // NW
    nchunk = bpw // C

    mesh = plsc.VectorSubcoreMesh(core_axis_name="c", subcore_axis_name="s")

    @functools.partial(
        pl.kernel,
        mesh=mesh,
        compiler_params=pltpu.CompilerParams(use_tc_tiling_on_sc=False),
        out_type=jax.ShapeDtypeStruct((n, NUM_EMB), jnp.float32),
        scratch_types=[
            pltpu.VMEM((C,), jnp.int32),
            pltpu.VMEM((C,), jnp.int32),
            pltpu.VMEM((NG, G), jnp.int32),
            pltpu.VMEM((C, NUM_EMB), jnp.float32),
            pltpu.SemaphoreType.DMA,
        ],
    )
    def body(off_hbm, msk_hbm, t_hbm, out_hbm, off_v, msk_v, d_v, rows_v, gsem):
        cid = lax.axis_index("c")
        sid = lax.axis_index("s")
        base = (sid * NC + cid) * bpw

        def chunk(i, carry):
            cb = base + i * C
            pltpu.sync_copy(off_hbm.at[pl.ds(cb, C)], off_v)
            pltpu.sync_copy(msk_hbm.at[pl.ds(cb, C)], msk_v)

            for g in range(NG):
                for j in range(G // L):
                    off = off_v[pl.ds((g * G + j * L), L)]
                    m = msk_v[pl.ds((g * G + j * L), L)]
                    t = jnp.minimum(jnp.maximum(off + MAX_REL, 0), 2 * MAX_REL)
                    d_v[g, pl.ds(j * L, L)] = jnp.where(m == 0, 2 * MAX_REL + 1, t)

            for g in range(NG):
                pltpu.async_copy(t_hbm.at[d_v.at[g]], rows_v.at[pl.ds(g * G, G)], gsem)
            pltpu.make_async_copy(out_hbm.at[pl.ds(cb, C)], rows_v, gsem).wait()

            pltpu.sync_copy(rows_v, out_hbm.at[pl.ds(cb, C)])
            return carry

        lax.fori_loop(0, nchunk, chunk, 0)

    return body


def kernel(offset, mask, W, b):
    bsz, seq, k = offset.shape
    n = bsz * seq * k
    off = offset.reshape(n).astype(jnp.int32)
    msk = mask.reshape(n).astype(jnp.int32)
    T = _fold_bias(W.astype(jnp.float32), b.astype(jnp.float32))
    out = _make_sc_lookup(n)(off, msk, T)
    return out.reshape(bsz, seq, k, NUM_EMB)


# speedup vs baseline: 4.3917x; 4.3917x over previous
import functools

import jax
import jax.numpy as jnp
from jax import lax
from jax.experimental import pallas as pl
from jax.experimental.pallas import tpu as pltpu
from jax.experimental.pallas import tpu_sc as plsc

MAX_REL = 32
NUM_EMB = 16
NUM_CLASSES = 2 * MAX_REL + 1 + 1

NC = 2
NS = 16
NW = NC * NS
L = 16

KD = 48
LANES = 128
R = 32
CPOS = R * KD


def _prep_body(off_ref, msk_ref, d_ref):
    off = off_ref[0]
    m = msk_ref[0]
    t = jnp.clip(off + MAX_REL, 0, 2 * MAX_REL)
    d = jnp.where(m == 0, 2 * MAX_REL + 1, t).astype(jnp.int32)
    d_ref[:, :KD] = d
    d_ref[:, KD:] = jnp.zeros((d.shape[0], LANES - KD), jnp.int32)


def _prep_d(offset, mask):
    bsz, seq, k = offset.shape
    rb = 256
    grid = (bsz, seq // rb)
    return pl.pallas_call(
        _prep_body,
        grid=grid,
        in_specs=[
            pl.BlockSpec((1, rb, k), lambda b, i: (b, i, 0)),
            pl.BlockSpec((1, rb, k), lambda b, i: (b, i, 0)),
        ],
        out_specs=pl.BlockSpec((rb, LANES), lambda b, i: (b * (seq // rb) + i, 0)),
        out_shape=jax.ShapeDtypeStruct((bsz * seq, LANES), jnp.int32),
    )(offset, mask)


def _bias_body(w_ref, b_ref, t_ref):
    t_ref[...] = w_ref[...] + b_ref[...]


def _fold_bias(W, b):
    return pl.pallas_call(
        _bias_body,
        out_shape=jax.ShapeDtypeStruct((NUM_CLASSES, NUM_EMB), jnp.float32),
    )(W, b.reshape(1, NUM_EMB))


def _make_sc_lookup(nrows):
    rpw = nrows // NW
    nchunk = rpw // R
    npos = nrows * KD

    mesh = plsc.VectorSubcoreMesh(core_axis_name="c", subcore_axis_name="s")

    @functools.partial(
        pl.kernel,
        mesh=mesh,
        compiler_params=pltpu.CompilerParams(
            use_tc_tiling_on_sc=False, needs_layout_passes=False
        ),
        out_type=jax.ShapeDtypeStruct((npos, NUM_EMB), jnp.float32),
        scratch_types=[
            pltpu.VMEM((NUM_CLASSES, NUM_EMB), jnp.float32),
            pltpu.VMEM((R, LANES), jnp.int32),
            pltpu.VMEM((R, LANES), jnp.int32),
            pltpu.VMEM((CPOS, NUM_EMB), jnp.float32),
            pltpu.VMEM((CPOS, NUM_EMB), jnp.float32),
            pltpu.SemaphoreType.DMA,
            pltpu.SemaphoreType.DMA,
            pltpu.SemaphoreType.DMA,
            pltpu.SemaphoreType.DMA,
        ],
    )
    def body(d_hbm, t_hbm, out_hbm, t_v, in0, in1, rows0, rows1, is0, is1, os0, os1):
        cid = lax.axis_index("c")
        sid = lax.axis_index("s")
        wid = sid * NC + cid
        row_base = wid * rpw

        pltpu.sync_copy(t_hbm, t_v)

        ins = [in0, in1]
        rows = [rows0, rows1]
        isems = [is0, is1]
        osems = [os0, os1]

        iota = lax.iota(jnp.int32, L)
        cvecs = [jnp.full((L,), c, jnp.int32) for c in range(NUM_EMB)]

        def start_in(i):
            s = i % 2
            return pltpu.async_copy(
                d_hbm.at[pl.ds(row_base + i * R, R)],
                ins[s],
                isems[s],
            )

        def start_out(i):
            s = i % 2
            return pltpu.async_copy(
                rows[s],
                out_hbm.at[pl.ds((row_base + i * R) * KD, CPOS)],
                osems[s],
            )

        in_d = [None, None]
        out_d = [None, None]
        in_d[0] = start_in(0)

        for i in range(nchunk):
            s = i % 2
            if i + 1 < nchunk:
                in_d[1 - s] = start_in(i + 1)
            in_d[s].wait()
            if out_d[s] is not None:
                out_d[s].wait()

            ibuf = ins[s]
            obuf = rows[s]

            def do_row(r, carry):
                for p in range(KD // L):
                    dvec = ibuf[r, pl.ds(p * L, L)]
                    st0 = iota + (r * KD + p * L)
                    for c in range(NUM_EMB):
                        col = plsc.load_gather(t_v, [dvec, cvecs[c]])
                        plsc.store_scatter(obuf, [st0, cvecs[c]], col)
                return carry

            lax.fori_loop(0, R, do_row, 0)

            out_d[s] = start_out(i)

        out_d[(nchunk - 1) % 2].wait()
        out_d[nchunk % 2].wait()

    return body


def kernel(offset, mask, W, b):
    bsz, seq, k = offset.shape
    assert k == KD
    d = _prep_d(offset.astype(jnp.int32), mask.astype(jnp.int32))
    T = _fold_bias(W.astype(jnp.float32), b.astype(jnp.float32))
    out = _make_sc_lookup(bsz * seq)(d, T)
    return out.reshape(bsz, seq, k, NUM_EMB)


# speedup vs baseline: 5.2702x; 1.2001x over previous
import functools

import jax
import jax.numpy as jnp
from jax import lax
from jax.experimental import pallas as pl
from jax.experimental.pallas import tpu as pltpu
from jax.experimental.pallas import tpu_sc as plsc

MAX_REL = 32
NUM_EMB = 16
NUM_CLASSES = 2 * MAX_REL + 1 + 1

NC = 2
NS = 16
NW = NC * NS
L = 16

KD = 48
LANES = 128
R = 32
CPOS = R * KD


def _prep_body(off_ref, msk_ref, d_ref):
    off = off_ref[0]
    m = msk_ref[0]
    t = jnp.clip(off + MAX_REL, 0, 2 * MAX_REL)
    d = jnp.where(m == 0, 2 * MAX_REL + 1, t).astype(jnp.int32)
    d_ref[:, :KD] = d
    d_ref[:, KD:] = jnp.zeros((d.shape[0], LANES - KD), jnp.int32)


def _prep_d(offset, mask):
    bsz, seq, k = offset.shape
    rb = 256
    grid = (bsz, seq // rb)
    return pl.pallas_call(
        _prep_body,
        grid=grid,
        in_specs=[
            pl.BlockSpec((1, rb, k), lambda b, i: (b, i, 0)),
            pl.BlockSpec((1, rb, k), lambda b, i: (b, i, 0)),
        ],
        out_specs=pl.BlockSpec((rb, LANES), lambda b, i: (b * (seq // rb) + i, 0)),
        out_shape=jax.ShapeDtypeStruct((bsz * seq, LANES), jnp.int32),
    )(offset, mask)


def _bias_body(w_ref, b_ref, t_ref):
    t_ref[...] = w_ref[...] + b_ref[...]


def _fold_bias(W, b):
    return pl.pallas_call(
        _bias_body,
        out_shape=jax.ShapeDtypeStruct((NUM_CLASSES, NUM_EMB), jnp.float32),
    )(W, b.reshape(1, NUM_EMB))


def _make_sc_lookup(nrows):
    rpw = nrows // NW
    nchunk = rpw // R
    nout = nrows * KD * NUM_EMB
    cout = CPOS * NUM_EMB

    mesh = plsc.VectorSubcoreMesh(core_axis_name="c", subcore_axis_name="s")

    @functools.partial(
        pl.kernel,
        mesh=mesh,
        compiler_params=pltpu.CompilerParams(
            use_tc_tiling_on_sc=False, needs_layout_passes=False
        ),
        out_type=jax.ShapeDtypeStruct((nout,), jnp.float32),
        scratch_types=[
            pltpu.VMEM((NUM_CLASSES * NUM_EMB,), jnp.float32),
            pltpu.VMEM((R, LANES), jnp.int32),
            pltpu.VMEM((R, LANES), jnp.int32),
            pltpu.VMEM((cout,), jnp.float32),
            pltpu.VMEM((cout,), jnp.float32),
            pltpu.SemaphoreType.DMA,
            pltpu.SemaphoreType.DMA,
            pltpu.SemaphoreType.DMA,
            pltpu.SemaphoreType.DMA,
        ],
    )
    def body(d_hbm, t_hbm, out_hbm, t_v, in0, in1, rows0, rows1, is0, is1, os0, os1):
        cid = lax.axis_index("c")
        sid = lax.axis_index("s")
        wid = sid * NC + cid
        row_base = wid * rpw

        pltpu.sync_copy(t_hbm, t_v)

        ins = [in0, in1]
        rows = [rows0, rows1]
        isems = [is0, is1]
        osems = [os0, os1]

        iota16 = lax.iota(jnp.int32, L) * NUM_EMB

        def start_in(i):
            s = i % 2
            return pltpu.async_copy(
                d_hbm.at[pl.ds(row_base + i * R, R)],
                ins[s],
                isems[s],
            )

        def start_out(i):
            s = i % 2
            return pltpu.async_copy(
                rows[s],
                out_hbm.at[pl.ds((row_base + i * R) * KD * NUM_EMB, cout)],
                osems[s],
            )

        in_d = [None, None]
        out_d = [None, None]
        in_d[0] = start_in(0)

        for i in range(nchunk):
            s = i % 2
            if i + 1 < nchunk:
                in_d[1 - s] = start_in(i + 1)
            in_d[s].wait()
            if out_d[s] is not None:
                out_d[s].wait()

            ibuf = ins[s]
            obuf = rows[s]

            def do_row(r, carry):
                rb = r * (KD * NUM_EMB)
                for p in range(KD // L):
                    dvec16 = ibuf[r, pl.ds(p * L, L)] * NUM_EMB
                    st0 = iota16 + (rb + p * L * NUM_EMB)
                    cols = [
                        plsc.load_gather(t_v, [dvec16 + c])
                        for c in range(NUM_EMB)
                    ]
                    for c in range(NUM_EMB):
                        plsc.store_scatter(obuf, [st0 + c], cols[c])
                return carry

            lax.fori_loop(0, R, do_row, 0)

            out_d[s] = start_out(i)

        out_d[(nchunk - 1) % 2].wait()
        out_d[nchunk % 2].wait()

    return body


def kernel(offset, mask, W, b):
    bsz, seq, k = offset.shape
    assert k == KD
    d = _prep_d(offset.astype(jnp.int32), mask.astype(jnp.int32))
    T = _fold_bias(W.astype(jnp.float32), b.astype(jnp.float32))
    out = _make_sc_lookup(bsz * seq)(d, T.reshape(NUM_CLASSES * NUM_EMB))
    return out.reshape(bsz, seq, k, NUM_EMB)


# speedup vs baseline: 25.3355x; 4.8073x over previous
import functools

import jax
import jax.numpy as jnp
from jax import lax
from jax.experimental import pallas as pl
from jax.experimental.pallas import tpu as pltpu
from jax.experimental.pallas import tpu_sc as plsc

MAX_REL = 32
NUM_EMB = 16
NUM_CLASSES = 2 * MAX_REL + 1 + 1

NC = 2
NS = 16
NW = NC * NS
L = 16

SEQ = 2048
KD = 48
PLANE = NUM_EMB * SEQ
STILE = SEQ // 128


def _prep_body(off_ref, msk_ref, d_ref):
    off = off_ref[0]
    m = msk_ref[0]
    t = jnp.clip(off + MAX_REL, 0, 2 * MAX_REL)
    d_ref[...] = jnp.where(m == 0, 2 * MAX_REL + 1, t).astype(jnp.int32)


def _prep_d(off_t, msk_t):
    bsz, k, seq = off_t.shape
    return pl.pallas_call(
        _prep_body,
        grid=(bsz,),
        in_specs=[
            pl.BlockSpec((1, k, seq), lambda b: (b, 0, 0)),
            pl.BlockSpec((1, k, seq), lambda b: (b, 0, 0)),
        ],
        out_specs=pl.BlockSpec((k, seq), lambda b: (b, 0)),
        out_shape=jax.ShapeDtypeStruct((bsz * k, seq), jnp.int32),
    )(off_t, msk_t)


def _bias_body(w_ref, b_ref, t_ref):
    t_ref[...] = w_ref[...] + b_ref[...]


def _fold_bias(W, b):
    return pl.pallas_call(
        _bias_body,
        out_shape=jax.ShapeDtypeStruct((NUM_CLASSES, NUM_EMB), jnp.float32),
    )(W, b.reshape(1, NUM_EMB))


def _make_sc_lookup(nplanes):
    ppw = nplanes // NW

    mesh = plsc.VectorSubcoreMesh(core_axis_name="c", subcore_axis_name="s")

    @functools.partial(
        pl.kernel,
        mesh=mesh,
        compiler_params=pltpu.CompilerParams(
            use_tc_tiling_on_sc=False, needs_layout_passes=False
        ),
        out_type=jax.ShapeDtypeStruct((nplanes * PLANE,), jnp.float32),
        scratch_types=[
            pltpu.VMEM((NUM_CLASSES * NUM_EMB,), jnp.float32),
            pltpu.VMEM((SEQ,), jnp.int32),
            pltpu.VMEM((SEQ,), jnp.int32),
            pltpu.VMEM((PLANE,), jnp.float32),
            pltpu.VMEM((PLANE,), jnp.float32),
            pltpu.SemaphoreType.DMA,
            pltpu.SemaphoreType.DMA,
            pltpu.SemaphoreType.DMA,
            pltpu.SemaphoreType.DMA,
        ],
    )
    def body(d_hbm, t_hbm, out_hbm, t_v, in0, in1, pl0, pl1, is0, is1, os0, os1):
        cid = lax.axis_index("c")
        sid = lax.axis_index("s")
        wid = sid * NC + cid
        plane_base = wid * ppw

        pltpu.sync_copy(t_hbm, t_v)

        ins = [in0, in1]
        obufs = [pl0, pl1]
        isems = [is0, is1]
        osems = [os0, os1]

        def start_in(i):
            s = i % 2
            return pltpu.async_copy(
                d_hbm.at[plane_base + i], ins[s], isems[s]
            )

        def start_out(i):
            s = i % 2
            return pltpu.async_copy(
                obufs[s],
                out_hbm.at[pl.ds((plane_base + i) * PLANE, PLANE)],
                osems[s],
            )

        in_d = [None, None]
        out_d = [None, None]
        in_d[0] = start_in(0)

        for i in range(ppw):
            s = i % 2
            if i + 1 < ppw:
                in_d[1 - s] = start_in(i + 1)
            in_d[s].wait()
            if out_d[s] is not None:
                out_d[s].wait()

            ibuf = ins[s]
            obuf = obufs[s]

            def do_stile(tc, carry):
                tb = tc * 128
                for gs in range(8):
                    dvec16 = ibuf[pl.ds(tc * 128 + gs * L, L)] * NUM_EMB
                    cols = [
                        plsc.load_gather(t_v, [dvec16 + c])
                        for c in range(NUM_EMB)
                    ]
                    for c in range(NUM_EMB):
                        o = (c // 8) * (STILE * 1024) + (c % 8) * 128 + gs * L
                        obuf[pl.ds(tb * 8 + o, L)] = cols[c]
                return carry

            lax.fori_loop(0, STILE, do_stile, 0)

            out_d[s] = start_out(i)

        out_d[(ppw - 1) % 2].wait()
        out_d[ppw % 2].wait()

    return body


def kernel(offset, mask, W, b):
    bsz, seq, k = offset.shape
    assert seq == SEQ and k == KD
    off_t = offset.astype(jnp.int32).transpose(0, 2, 1)
    msk_t = mask.astype(jnp.int32).transpose(0, 2, 1)
    d_t = _prep_d(off_t, msk_t)
    T = _fold_bias(W.astype(jnp.float32), b.astype(jnp.float32))
    out = _make_sc_lookup(bsz * k)(d_t, T.reshape(NUM_CLASSES * NUM_EMB))
    out6 = out.reshape(bsz, k, 2, STILE, 8, 128)
    return out6.transpose(0, 3, 5, 1, 2, 4).reshape(bsz, seq, k, NUM_EMB)


# speedup vs baseline: 25.4105x; 1.0030x over previous
import functools

import jax
import jax.numpy as jnp
from jax import lax
from jax.experimental import pallas as pl
from jax.experimental.pallas import tpu as pltpu
from jax.experimental.pallas import tpu_sc as plsc

MAX_REL = 32
NUM_EMB = 16
NUM_CLASSES = 2 * MAX_REL + 1 + 1

NC = 2
NS = 16
NW = NC * NS
L = 16

SEQ = 2048
KD = 48
PLANE = NUM_EMB * SEQ
STILE = SEQ // 128


def _bias_body(w_ref, b_ref, t_ref):
    t_ref[...] = w_ref[...] + b_ref[...]


def _fold_bias(W, b):
    return pl.pallas_call(
        _bias_body,
        out_shape=jax.ShapeDtypeStruct((NUM_CLASSES, NUM_EMB), jnp.float32),
    )(W, b.reshape(1, NUM_EMB))


def _make_sc_lookup(nplanes):
    ppw = nplanes // NW

    mesh = plsc.VectorSubcoreMesh(core_axis_name="c", subcore_axis_name="s")

    @functools.partial(
        pl.kernel,
        mesh=mesh,
        compiler_params=pltpu.CompilerParams(
            use_tc_tiling_on_sc=False, needs_layout_passes=False
        ),
        out_type=jax.ShapeDtypeStruct((nplanes * PLANE,), jnp.float32),
        scratch_types=[
            pltpu.VMEM((NUM_CLASSES * NUM_EMB,), jnp.float32),
            pltpu.VMEM((SEQ,), jnp.int32),
            pltpu.VMEM((SEQ,), jnp.int32),
            pltpu.VMEM((SEQ,), jnp.int32),
            pltpu.VMEM((SEQ,), jnp.int32),
            pltpu.VMEM((PLANE,), jnp.float32),
            pltpu.VMEM((PLANE,), jnp.float32),
            pltpu.SemaphoreType.DMA,
            pltpu.SemaphoreType.DMA,
            pltpu.SemaphoreType.DMA,
            pltpu.SemaphoreType.DMA,
        ],
    )
    def body(off_hbm, msk_hbm, t_hbm, out_hbm,
             t_v, of0, of1, mk0, mk1, pl0, pl1, is0, is1, os0, os1):
        cid = lax.axis_index("c")
        sid = lax.axis_index("s")
        wid = sid * NC + cid
        plane_base = wid * ppw

        pltpu.sync_copy(t_hbm, t_v)

        offs = [of0, of1]
        msks = [mk0, mk1]
        obufs = [pl0, pl1]
        isems = [is0, is1]
        osems = [os0, os1]

        def start_in(i):
            s = i % 2
            p = plane_base + i
            b = p // KD
            k = p % KD
            a = pltpu.async_copy(off_hbm.at[b, k], offs[s], isems[s])
            c = pltpu.async_copy(msk_hbm.at[b, k], msks[s], isems[s])
            return (a, c)

        def start_out(i):
            s = i % 2
            return pltpu.async_copy(
                obufs[s],
                out_hbm.at[pl.ds((plane_base + i) * PLANE, PLANE)],
                osems[s],
            )

        in_d = [None, None]
        out_d = [None, None]
        in_d[0] = start_in(0)

        for i in range(ppw):
            s = i % 2
            if i + 1 < ppw:
                in_d[1 - s] = start_in(i + 1)
            in_d[s][0].wait()
            in_d[s][1].wait()
            if out_d[s] is not None:
                out_d[s].wait()

            obuf = obufs[s]
            ofb = offs[s]
            mkb = msks[s]

            def do_stile(tc, carry):
                tb = tc * 128
                for gs in range(8):
                    off = ofb[pl.ds(tc * 128 + gs * L, L)]
                    m = mkb[pl.ds(tc * 128 + gs * L, L)]
                    t = jnp.minimum(off + MAX_REL, 2 * MAX_REL)
                    d = jnp.where(m == 0, 2 * MAX_REL + 1, t)
                    dvec16 = d * NUM_EMB
                    cols = [
                        plsc.load_gather(t_v, [dvec16 + c])
                        for c in range(NUM_EMB)
                    ]
                    for c in range(NUM_EMB):
                        o = (c // 8) * (STILE * 1024) + (c % 8) * 128 + gs * L
                        obuf[pl.ds(tb * 8 + o, L)] = cols[c]
                return carry

            lax.fori_loop(0, STILE, do_stile, 0)

            out_d[s] = start_out(i)

        out_d[(ppw - 1) % 2].wait()
        out_d[ppw % 2].wait()

    return body


def kernel(offset, mask, W, b):
    bsz, seq, k = offset.shape
    assert seq == SEQ and k == KD
    off_t = offset.astype(jnp.int32).transpose(0, 2, 1)
    msk_t = mask.astype(jnp.int32).transpose(0, 2, 1)
    T = _fold_bias(W.astype(jnp.float32), b.astype(jnp.float32))
    out = _make_sc_lookup(bsz * k)(off_t, msk_t, T.reshape(NUM_CLASSES * NUM_EMB))
    out6 = out.reshape(bsz, k, 2, STILE, 8, 128)
    return out6.transpose(0, 3, 5, 1, 2, 4).reshape(bsz, seq, k, NUM_EMB)


# speedup vs baseline: 27.1040x; 1.0666x over previous
import functools

import jax
import jax.numpy as jnp
from jax import lax
from jax.experimental import pallas as pl
from jax.experimental.pallas import tpu as pltpu
from jax.experimental.pallas import tpu_sc as plsc

MAX_REL = 32
NUM_EMB = 16
NUM_CLASSES = 2 * MAX_REL + 1 + 1

NC = 2
NS = 16
NW = NC * NS
L = 16

SEQ = 2048
KD = 48
PLANE = NUM_EMB * SEQ
STILE = SEQ // 128


def _bias_body(w_ref, b_ref, t_ref):
    t_ref[...] = w_ref[...] + b_ref[...]


def _fold_bias(W, b):
    return pl.pallas_call(
        _bias_body,
        out_shape=jax.ShapeDtypeStruct((NUM_CLASSES, NUM_EMB), jnp.float32),
    )(W, b.reshape(1, NUM_EMB))


def _make_sc_lookup(nplanes):
    ppw = nplanes // NW

    mesh = plsc.VectorSubcoreMesh(core_axis_name="c", subcore_axis_name="s")

    @functools.partial(
        pl.kernel,
        mesh=mesh,
        compiler_params=pltpu.CompilerParams(
            use_tc_tiling_on_sc=False, needs_layout_passes=False
        ),
        out_type=jax.ShapeDtypeStruct((nplanes * PLANE,), jnp.float32),
        scratch_types=[
            pltpu.VMEM((NUM_CLASSES * NUM_EMB,), jnp.float32),
            pltpu.VMEM((SEQ,), jnp.int32),
            pltpu.VMEM((SEQ,), jnp.int32),
            pltpu.VMEM((SEQ,), jnp.int32),
            pltpu.VMEM((SEQ,), jnp.int32),
            pltpu.VMEM((PLANE,), jnp.float32),
            pltpu.VMEM((PLANE,), jnp.float32),
            pltpu.SemaphoreType.DMA,
            pltpu.SemaphoreType.DMA,
            pltpu.SemaphoreType.DMA,
            pltpu.SemaphoreType.DMA,
        ],
    )
    def body(off_hbm, msk_hbm, t_hbm, out_hbm,
             t_v, of0, of1, mk0, mk1, pl0, pl1, is0, is1, os0, os1):
        cid = lax.axis_index("c")
        sid = lax.axis_index("s")
        wid = sid * NC + cid
        plane_base = wid * ppw

        pltpu.sync_copy(t_hbm, t_v)

        offs = [of0, of1]
        msks = [mk0, mk1]
        obufs = [pl0, pl1]
        isems = [is0, is1]
        osems = [os0, os1]

        def start_in(i):
            s = i % 2
            p = plane_base + i
            b = p // KD
            k = p % KD
            a = pltpu.async_copy(off_hbm.at[b, k], offs[s], isems[s])
            c = pltpu.async_copy(msk_hbm.at[b, k], msks[s], isems[s])
            return (a, c)

        def start_out(i):
            s = i % 2
            return pltpu.async_copy(
                obufs[s],
                out_hbm.at[pl.ds((plane_base + i) * PLANE, PLANE)],
                osems[s],
            )

        in_d = [None, None]
        out_d = [None, None]
        in_d[0] = start_in(0)

        for i in range(ppw):
            s = i % 2
            if i + 1 < ppw:
                in_d[1 - s] = start_in(i + 1)
            in_d[s][0].wait()
            in_d[s][1].wait()
            if out_d[s] is not None:
                out_d[s].wait()

            obuf = obufs[s]
            ofb = offs[s]
            mkb = msks[s]

            @plsc.parallel_loop(0, STILE, unroll=2)
            def do_stile(tc):
                tb = tc * 128
                for gs in range(8):
                    off = ofb[pl.ds(tc * 128 + gs * L, L)]
                    m = mkb[pl.ds(tc * 128 + gs * L, L)]
                    t = jnp.minimum(off + MAX_REL, 2 * MAX_REL)
                    d = jnp.where(m == 0, 2 * MAX_REL + 1, t)
                    dvec16 = d * NUM_EMB
                    cols = [
                        plsc.load_gather(t_v, [dvec16 + c])
                        for c in range(NUM_EMB)
                    ]
                    for c in range(NUM_EMB):
                        o = (c // 8) * (STILE * 1024) + (c % 8) * 128 + gs * L
                        obuf[pl.ds(tb * 8 + o, L)] = cols[c]

            out_d[s] = start_out(i)

        out_d[(ppw - 1) % 2].wait()
        out_d[ppw % 2].wait()

    return body


def kernel(offset, mask, W, b):
    bsz, seq, k = offset.shape
    assert seq == SEQ and k == KD
    off_t = offset.astype(jnp.int32).transpose(0, 2, 1)
    msk_t = mask.astype(jnp.int32).transpose(0, 2, 1)
    T = _fold_bias(W.astype(jnp.float32), b.astype(jnp.float32))
    out = _make_sc_lookup(bsz * k)(off_t, msk_t, T.reshape(NUM_CLASSES * NUM_EMB))
    out6 = out.reshape(bsz, k, 2, STILE, 8, 128)
    return out6.transpose(0, 3, 5, 1, 2, 4).reshape(bsz, seq, k, NUM_EMB)


# speedup vs baseline: 47.0791x; 1.7370x over previous
import functools

import jax
import jax.numpy as jnp
from jax import lax
from jax.experimental import pallas as pl
from jax.experimental.pallas import tpu as pltpu
from jax.experimental.pallas import tpu_sc as plsc

MAX_REL = 32
NUM_EMB = 16
NUM_CLASSES = 2 * MAX_REL + 1 + 1

NC = 2
NS = 16
NW = NC * NS
L = 16

SEQ = 2048
KD = 48
PLANE = NUM_EMB * SEQ
STILE = SEQ // 128


def _bias_body(w_ref, b_ref, t_ref):
    t_ref[...] = w_ref[...] + b_ref[...]


def _fold_bias(W, b):
    return pl.pallas_call(
        _bias_body,
        out_shape=jax.ShapeDtypeStruct((NUM_EMB, NUM_CLASSES), jnp.float32),
    )(W.T, b.reshape(NUM_EMB, 1))


def _make_sc_lookup(nplanes):
    ppw = nplanes // NW

    mesh = plsc.VectorSubcoreMesh(core_axis_name="c", subcore_axis_name="s")

    @functools.partial(
        pl.kernel,
        mesh=mesh,
        compiler_params=pltpu.CompilerParams(
            use_tc_tiling_on_sc=False, needs_layout_passes=False
        ),
        out_type=jax.ShapeDtypeStruct((nplanes * PLANE,), jnp.float32),
        scratch_types=[
            pltpu.VMEM((NUM_CLASSES * NUM_EMB,), jnp.float32),
            pltpu.VMEM((SEQ,), jnp.int32),
            pltpu.VMEM((SEQ,), jnp.int32),
            pltpu.VMEM((SEQ,), jnp.int32),
            pltpu.VMEM((SEQ,), jnp.int32),
            pltpu.VMEM((PLANE,), jnp.float32),
            pltpu.VMEM((PLANE,), jnp.float32),
            pltpu.SemaphoreType.DMA,
            pltpu.SemaphoreType.DMA,
            pltpu.SemaphoreType.DMA,
            pltpu.SemaphoreType.DMA,
        ],
    )
    def body(off_hbm, msk_hbm, t_hbm, out_hbm,
             t_v, of0, of1, mk0, mk1, pl0, pl1, is0, is1, os0, os1):
        cid = lax.axis_index("c")
        sid = lax.axis_index("s")
        wid = sid * NC + cid
        plane_base = wid * ppw

        pltpu.sync_copy(t_hbm, t_v)

        offs = [of0, of1]
        msks = [mk0, mk1]
        obufs = [pl0, pl1]
        isems = [is0, is1]
        osems = [os0, os1]

        def start_in(i):
            s = i % 2
            p = plane_base + i
            b = p // KD
            k = p % KD
            a = pltpu.async_copy(off_hbm.at[b, k], offs[s], isems[s])
            c = pltpu.async_copy(msk_hbm.at[b, k], msks[s], isems[s])
            return (a, c)

        def start_out(i):
            s = i % 2
            return pltpu.async_copy(
                obufs[s],
                out_hbm.at[pl.ds((plane_base + i) * PLANE, PLANE)],
                osems[s],
            )

        in_d = [None, None]
        out_d = [None, None]
        in_d[0] = start_in(0)

        for i in range(ppw):
            s = i % 2
            if i + 1 < ppw:
                in_d[1 - s] = start_in(i + 1)
            in_d[s][0].wait()
            in_d[s][1].wait()
            if out_d[s] is not None:
                out_d[s].wait()

            obuf = obufs[s]
            ofb = offs[s]
            mkb = msks[s]

            @plsc.parallel_loop(0, STILE, unroll=2)
            def do_stile(tc):
                tb = tc * 128
                for gs in range(8):
                    off = ofb[pl.ds(tc * 128 + gs * L, L)]
                    m = mkb[pl.ds(tc * 128 + gs * L, L)]
                    t = jnp.minimum(off + MAX_REL, 2 * MAX_REL)
                    d = jnp.where(m == 0, 2 * MAX_REL + 1, t)
                    cols = [
                        plsc.load_gather(t_v, [d + (NUM_CLASSES * c)])
                        for c in range(NUM_EMB)
                    ]
                    for c in range(NUM_EMB):
                        o = (c // 8) * (STILE * 1024) + (c % 8) * 128 + gs * L
                        obuf[pl.ds(tb * 8 + o, L)] = cols[c]

            out_d[s] = start_out(i)

        out_d[(ppw - 1) % 2].wait()
        out_d[ppw % 2].wait()

    return body


def kernel(offset, mask, W, b):
    bsz, seq, k = offset.shape
    assert seq == SEQ and k == KD
    off_t = offset.astype(jnp.int32).transpose(0, 2, 1)
    msk_t = mask.astype(jnp.int32).transpose(0, 2, 1)
    T = _fold_bias(W.astype(jnp.float32), b.astype(jnp.float32))
    out = _make_sc_lookup(bsz * k)(off_t, msk_t, T.reshape(NUM_CLASSES * NUM_EMB))
    out6 = out.reshape(bsz, k, 2, STILE, 8, 128)
    return out6.transpose(0, 3, 5, 1, 2, 4).reshape(bsz, seq, k, NUM_EMB)


# speedup vs baseline: 47.7107x; 1.0134x over previous
import functools

import jax
import jax.numpy as jnp
from jax import lax
from jax.experimental import pallas as pl
from jax.experimental.pallas import tpu as pltpu
from jax.experimental.pallas import tpu_sc as plsc

MAX_REL = 32
NUM_EMB = 16
NUM_CLASSES = 2 * MAX_REL + 1 + 1

NC = 2
NS = 16
NW = NC * NS
L = 16

SEQ = 2048
KD = 48
PLANE = NUM_EMB * SEQ
STILE = SEQ // 128


def _bias_body(w_ref, b_ref, t_ref):
    t_ref[...] = w_ref[...] + b_ref[...]


def _fold_bias(W, b):
    return pl.pallas_call(
        _bias_body,
        out_shape=jax.ShapeDtypeStruct((NUM_EMB, NUM_CLASSES), jnp.float32),
    )(W.T, b.reshape(NUM_EMB, 1))


def _make_sc_lookup(nplanes):
    ppw = nplanes // NW

    mesh = plsc.VectorSubcoreMesh(core_axis_name="c", subcore_axis_name="s")

    @functools.partial(
        pl.kernel,
        mesh=mesh,
        compiler_params=pltpu.CompilerParams(
            use_tc_tiling_on_sc=False, needs_layout_passes=False
        ),
        out_type=jax.ShapeDtypeStruct((nplanes * PLANE,), jnp.float32),
        scratch_types=[
            pltpu.VMEM((NUM_CLASSES * NUM_EMB * L,), jnp.float32),
            pltpu.VMEM((SEQ,), jnp.int32),
            pltpu.VMEM((SEQ,), jnp.int32),
            pltpu.VMEM((SEQ,), jnp.int32),
            pltpu.VMEM((SEQ,), jnp.int32),
            pltpu.VMEM((PLANE,), jnp.float32),
            pltpu.VMEM((PLANE,), jnp.float32),
            pltpu.SemaphoreType.DMA,
            pltpu.SemaphoreType.DMA,
            pltpu.SemaphoreType.DMA,
            pltpu.SemaphoreType.DMA,
        ],
    )
    def body(off_hbm, msk_hbm, t_hbm, out_hbm,
             t_v, of0, of1, mk0, mk1, pl0, pl1, is0, is1, os0, os1):
        cid = lax.axis_index("c")
        sid = lax.axis_index("s")
        wid = sid * NC + cid
        plane_base = wid * ppw

        pltpu.sync_copy(t_hbm, t_v)
        iota = lax.iota(jnp.int32, L)

        offs = [of0, of1]
        msks = [mk0, mk1]
        obufs = [pl0, pl1]
        isems = [is0, is1]
        osems = [os0, os1]

        def start_in(i):
            s = i % 2
            p = plane_base + i
            b = p // KD
            k = p % KD
            a = pltpu.async_copy(off_hbm.at[b, k], offs[s], isems[s])
            c = pltpu.async_copy(msk_hbm.at[b, k], msks[s], isems[s])
            return (a, c)

        def start_out(i):
            s = i % 2
            return pltpu.async_copy(
                obufs[s],
                out_hbm.at[pl.ds((plane_base + i) * PLANE, PLANE)],
                osems[s],
            )

        in_d = [None, None]
        out_d = [None, None]
        in_d[0] = start_in(0)

        for i in range(ppw):
            s = i % 2
            if i + 1 < ppw:
                in_d[1 - s] = start_in(i + 1)
            in_d[s][0].wait()
            in_d[s][1].wait()
            if out_d[s] is not None:
                out_d[s].wait()

            obuf = obufs[s]
            ofb = offs[s]
            mkb = msks[s]

            @plsc.parallel_loop(0, STILE, unroll=2)
            def do_stile(tc):
                tb = tc * 128
                for gs in range(8):
                    off = ofb[pl.ds(tc * 128 + gs * L, L)]
                    m = mkb[pl.ds(tc * 128 + gs * L, L)]
                    t = jnp.minimum(off + MAX_REL, 2 * MAX_REL)
                    d = jnp.where(m == 0, 2 * MAX_REL + 1, t)
                    dsc = d * L + iota
                    cols = [
                        plsc.load_gather(t_v, [dsc + (NUM_CLASSES * L * c)])
                        for c in range(NUM_EMB)
                    ]
                    for c in range(NUM_EMB):
                        o = (c // 8) * (STILE * 1024) + (c % 8) * 128 + gs * L
                        obuf[pl.ds(tb * 8 + o, L)] = cols[c]

            out_d[s] = start_out(i)

        out_d[(ppw - 1) % 2].wait()
        out_d[ppw % 2].wait()

    return body


def kernel(offset, mask, W, b):
    bsz, seq, k = offset.shape
    assert seq == SEQ and k == KD
    off_t = offset.astype(jnp.int32).transpose(0, 2, 1)
    msk_t = mask.astype(jnp.int32).transpose(0, 2, 1)
    T = _fold_bias(W.astype(jnp.float32), b.astype(jnp.float32))
    T_rep = jnp.broadcast_to(
        T.reshape(NUM_CLASSES * NUM_EMB, 1), (NUM_CLASSES * NUM_EMB, L)
    ).reshape(NUM_CLASSES * NUM_EMB * L)
    out = _make_sc_lookup(bsz * k)(off_t, msk_t, T_rep)
    out6 = out.reshape(bsz, k, 2, STILE, 8, 128)
    return out6.transpose(0, 3, 5, 1, 2, 4).reshape(bsz, seq, k, NUM_EMB)


# speedup vs baseline: 47.7690x; 1.0012x over previous
import functools

import jax
import jax.numpy as jnp
from jax import lax
from jax.experimental import pallas as pl
from jax.experimental.pallas import tpu as pltpu
from jax.experimental.pallas import tpu_sc as plsc

MAX_REL = 32
NUM_EMB = 16
NUM_CLASSES = 2 * MAX_REL + 1 + 1

NC = 2
NS = 16
NW = NC * NS
L = 16

SEQ = 2048
KD = 48
PLANE = NUM_EMB * SEQ
STILE = SEQ // 128


def _bias_body(w_ref, b_ref, t_ref):
    t_ref[...] = w_ref[...] + b_ref[...]


def _fold_bias(W, b):
    return pl.pallas_call(
        _bias_body,
        out_shape=jax.ShapeDtypeStruct((NUM_EMB, NUM_CLASSES), jnp.float32),
    )(W.T, b.reshape(NUM_EMB, 1))


def _make_sc_lookup(nplanes):
    ppw = nplanes // NW

    mesh = plsc.VectorSubcoreMesh(core_axis_name="c", subcore_axis_name="s")

    @functools.partial(
        pl.kernel,
        mesh=mesh,
        compiler_params=pltpu.CompilerParams(
            use_tc_tiling_on_sc=False, needs_layout_passes=False
        ),
        out_type=jax.ShapeDtypeStruct((nplanes * PLANE,), jnp.float32),
        scratch_types=[
            pltpu.VMEM((NUM_CLASSES * NUM_EMB * L,), jnp.float32),
            pltpu.VMEM((SEQ,), jnp.int32),
            pltpu.VMEM((SEQ,), jnp.int32),
            pltpu.VMEM((SEQ,), jnp.int32),
            pltpu.VMEM((SEQ,), jnp.int32),
            pltpu.VMEM((PLANE,), jnp.float32),
            pltpu.VMEM((PLANE,), jnp.float32),
            pltpu.SemaphoreType.DMA,
            pltpu.SemaphoreType.DMA,
            pltpu.SemaphoreType.DMA,
            pltpu.SemaphoreType.DMA,
        ],
    )
    def body(off_hbm, msk_hbm, t_hbm, out_hbm,
             t_v, of0, of1, mk0, mk1, pl0, pl1, is0, is1, os0, os1):
        cid = lax.axis_index("c")
        sid = lax.axis_index("s")
        wid = sid * NC + cid
        plane_base = wid * ppw

        pltpu.sync_copy(t_hbm, t_v)
        iota = lax.iota(jnp.int32, L)

        offs = [of0, of1]
        msks = [mk0, mk1]
        obufs = [pl0, pl1]
        isems = [is0, is1]
        osems = [os0, os1]

        def start_in(i):
            s = i % 2
            p = plane_base + i
            b = p // KD
            k = p % KD
            a = pltpu.async_copy(off_hbm.at[b, k], offs[s], isems[s])
            c = pltpu.async_copy(msk_hbm.at[b, k], msks[s], isems[s])
            return (a, c)

        def start_out(i):
            s = i % 2
            return pltpu.async_copy(
                obufs[s],
                out_hbm.at[pl.ds((plane_base + i) * PLANE, PLANE)],
                osems[s],
            )

        in_d = [None, None]
        out_d = [None, None]
        in_d[0] = start_in(0)

        for i in range(ppw):
            s = i % 2
            if i + 1 < ppw:
                in_d[1 - s] = start_in(i + 1)
            for dsc in in_d[s]:
                dsc.wait()
            if out_d[s] is not None:
                out_d[s].wait()

            obuf = obufs[s]
            ofb = offs[s]
            mkb = msks[s]

            @plsc.parallel_loop(0, STILE, unroll=2)
            def do_stile(tc):
                tb = tc * 128
                for gs in range(8):
                    off = ofb[pl.ds(tc * 128 + gs * L, L)]
                    m = mkb[pl.ds(tc * 128 + gs * L, L)]
                    t = jnp.minimum(off + MAX_REL, 2 * MAX_REL)
                    d = jnp.where(m == 0, 2 * MAX_REL + 1, t)
                    dsc = d * L + iota
                    cols = [
                        plsc.load_gather(t_v, [dsc + (NUM_CLASSES * L * c)])
                        for c in range(NUM_EMB)
                    ]
                    for c in range(NUM_EMB):
                        o = (c // 8) * (STILE * 1024) + (c % 8) * 128 + gs * L
                        obuf[pl.ds(tb * 8 + o, L)] = cols[c]

            out_d[s] = start_out(i)

        out_d[(ppw - 1) % 2].wait()
        out_d[ppw % 2].wait()

    return body


def kernel(offset, mask, W, b):
    bsz, seq, k = offset.shape
    assert seq == SEQ and k == KD
    off_t = offset.astype(jnp.int32).transpose(0, 2, 1)
    msk_t = mask.astype(jnp.int32).transpose(0, 2, 1)
    T = _fold_bias(W.astype(jnp.float32), b.astype(jnp.float32))
    T_rep = jnp.broadcast_to(
        T.reshape(NUM_CLASSES * NUM_EMB, 1), (NUM_CLASSES * NUM_EMB, L)
    ).reshape(NUM_CLASSES * NUM_EMB * L)
    out = _make_sc_lookup(bsz * k)(off_t, msk_t, T_rep)
    out6 = out.reshape(bsz, k, 2, STILE, 8, 128)
    return out6.transpose(0, 3, 5, 1, 2, 4).reshape(bsz, seq, k, NUM_EMB)
